# baseline (speedup 1.0000x reference)
import functools

import jax
import jax.numpy as jnp
import numpy as np
from jax import lax
from jax.experimental import pallas as pl
from jax.experimental.pallas import tpu as pltpu
from jax.experimental.pallas import tpu_sc as plsc

N = 10000
E = 320000
D = 128
A = 16
AVG_NUM_NEIGHBORS = 32

NC, NS = 2, 16
NW = NC * NS
CHUNK = 128
NCHUNKS = E // CHUNK
BASE = NCHUNKS // NW
REM = NCHUNKS % NW
RPT = 624

_sc_mesh = plsc.VectorSubcoreMesh(core_axis_name="c", subcore_axis_name="s")


NBUF = 3
LOOKAHEAD = 2
ZR = 8


@functools.partial(
    pl.kernel,
    out_type=jax.ShapeDtypeStruct((NC, N, D), jnp.float32),
    mesh=_sc_mesh,
    scratch_types=[
        pltpu.VMEM_SHARED((N, D), jnp.float32),
        [pltpu.VMEM((CHUNK,), jnp.int32) for _ in range(NBUF)],
        [pltpu.VMEM((CHUNK, D), jnp.float32) for _ in range(NBUF)],
        pltpu.VMEM((ZR, D), jnp.float32),
        [pltpu.SemaphoreType.DMA for _ in range(NBUF)],
        [pltpu.SemaphoreType.DMA for _ in range(NBUF)],
    ],
)
def _sc_scatter(msg_hbm, ei_hbm, out_hbm, acc, idxs, msgs, zbuf, lsems, ssems):
    cid = lax.axis_index("c")
    sid = lax.axis_index("s")
    wid = sid * NC + cid

    start = wid * BASE

    def load(k, b):
        e0 = (start + k) * CHUNK
        pltpu.async_copy(ei_hbm.at[1, pl.ds(e0, CHUNK)], idxs[b], lsems[b])
        pltpu.async_copy(msg_hbm.at[pl.ds(e0, CHUNK)], msgs[b], lsems[b])

    def wait_load(b):
        pltpu.make_async_copy(ei_hbm.at[1, pl.ds(0, CHUNK)], idxs[b], lsems[b]).wait()
        pltpu.make_async_copy(msg_hbm.at[pl.ds(0, CHUNK)], msgs[b], lsems[b]).wait()

    def wait_scatter(b):
        pltpu.make_async_copy(msgs[b], acc.at[idxs[b]], ssems[b]).wait()

    for b in range(LOOKAHEAD):
        load(b, b)

    for r in range(ZR):
        for c in range(D // 16):
            zbuf[r, pl.ds(c * 16, 16)] = jnp.zeros((16,), jnp.float32)

    nz = RPT // ZR + (ZR * 2 // ZR) * (sid == NS - 1)

    def zcopy(j, carry):
        pltpu.sync_copy(zbuf, acc.at[pl.ds(sid * RPT + j * ZR, ZR)])
        return carry

    lax.fori_loop(0, nz, zcopy, 0)
    plsc.subcore_barrier()

    def body(i, carry):
        for b in range(NBUF):
            k = NBUF * i + b
            wait_load(b)
            pltpu.async_copy(msgs[b], acc.at[idxs[b]], ssems[b], add=True)
            bn = (b + LOOKAHEAD) % NBUF

            @pl.when(k + LOOKAHEAD < BASE)
            def _():
                @pl.when(k >= 1)
                def _():
                    wait_scatter(bn)

                load(k + LOOKAHEAD, bn)

        return carry

    lax.fori_loop(0, BASE // NBUF, body, 0)

    for b in range(NBUF):
        wait_scatter(b)

    @pl.when(wid < REM)
    def _():
        e0 = (NW * BASE + wid) * CHUNK
        pltpu.sync_copy(ei_hbm.at[1, pl.ds(e0, CHUNK)], idxs[0])
        pltpu.sync_copy(msg_hbm.at[pl.ds(e0, CHUNK)], msgs[0])
        pltpu.sync_copy(msgs[0], acc.at[idxs[0]], add=True)

    plsc.subcore_barrier()

    @pl.when(sid < NS - 1)
    def _():
        pltpu.sync_copy(
            acc.at[pl.ds(sid * RPT, RPT)],
            out_hbm.at[cid, pl.ds(sid * RPT, RPT)],
        )

    @pl.when(sid == NS - 1)
    def _():
        pltpu.sync_copy(
            acc.at[pl.ds((NS - 1) * RPT, N - (NS - 1) * RPT)],
            out_hbm.at[cid, pl.ds((NS - 1) * RPT, N - (NS - 1) * RPT)],
        )


BLK = 400


def _sc_part_body(f_ref, a_ref, ws_ref, o_ref):
    f = f_ref[...]
    a = a_ref[...]
    acc = jnp.zeros((BLK, D), jnp.float32)
    for j in range(A):
        prod = (f * a[:, j][:, None]).astype(jnp.bfloat16)
        acc = acc + jnp.dot(prod, ws_ref[j], preferred_element_type=jnp.float32)
    o_ref[...] = acc


_sc_part_call = pl.pallas_call(
    _sc_part_body,
    grid=(N // BLK,),
    in_specs=[
        pl.BlockSpec((BLK, D), lambda i: (i, 0)),
        pl.BlockSpec((BLK, A), lambda i: (i, 0)),
        pl.BlockSpec((A, D, D), lambda i: (0, 0, 0)),
    ],
    out_specs=pl.BlockSpec((BLK, D), lambda i: (i, 0)),
    out_shape=jax.ShapeDtypeStruct((N, D), jnp.float32),
)


def _combine_body(p_ref, s_ref, wl_ref, o_ref):
    msg = p_ref[0] + p_ref[1]
    o_ref[...] = s_ref[...] + jnp.dot(
        msg, wl_ref[...], preferred_element_type=jnp.float32
    )


CBLK = 2000

_combine_call = pl.pallas_call(
    _combine_body,
    grid=(N // CBLK,),
    in_specs=[
        pl.BlockSpec((NC, CBLK, D), lambda i: (0, i, 0)),
        pl.BlockSpec((CBLK, D), lambda i: (i, 0)),
        pl.BlockSpec((D, D), lambda i: (0, 0)),
    ],
    out_specs=pl.BlockSpec((CBLK, D), lambda i: (i, 0)),
    out_shape=jax.ShapeDtypeStruct((N, D), jnp.float32),
)


@jax.jit
def kernel(node_feats, node_attrs, edge_message, edge_index, W_lin, W_sc):
    partial = _sc_scatter(edge_message, edge_index)
    wl_scaled = W_lin * np.float32(1.0 / np.sqrt(AVG_NUM_NEIGHBORS))
    wsc_t = jnp.transpose(W_sc, (1, 0, 2)).astype(jnp.bfloat16)
    sc_part = _sc_part_call(node_feats, node_attrs, wsc_t)
    return _combine_call(partial, sc_part, wl_scaled)
